# baseline (speedup 1.0000x reference)
import jax
import jax.numpy as jnp
from jax import lax
from jax.experimental import pallas as pl
from jax.experimental.pallas import tpu as pltpu
from jax.experimental.pallas import tpu_sc as plsc

HIDDEN = 768
BATCH = 4
SEQ = 2048
TOK = BATCH * SEQ

NC, NS = 2, 16
NW = NC * NS
PPW = SEQ // NW
C = 16
NPC = PPW // C
NSTEP = NPC * BATCH
NBUF = 4
GROUPS = HIDDEN // 16

_DNUMS = lax.GatherDimensionNumbers(
    offset_dims=(), collapsed_slice_dims=(0,), start_index_map=(0,))


def _embed_body(ids_hbm, tt_hbm, word_hbm, type_hbm, pos_hbm, out_hbm,
                widx_v, tidx_v, typ2_v, dt_v,
                w0, w1, w2, w3, p0, p1,
                sw0, sw1, sw2, sw3, sp0, sp1, so0, so1, so2, so3):
    wid = lax.axis_index("s") * NC + lax.axis_index("c")
    pbase = wid * PPW

    for b in range(BATCH):
        pltpu.sync_copy(ids_hbm.at[pl.ds(b * SEQ + pbase, PPW)],
                        widx_v.at[pl.ds(b * PPW, PPW)])
        pltpu.sync_copy(tt_hbm.at[pl.ds(b * SEQ + pbase, PPW)],
                        tidx_v.at[pl.ds(b * PPW, PPW)])
    pltpu.sync_copy(type_hbm, typ2_v)
    for g in range(GROUPS):
        sl = pl.ds(g * 16, 16)
        dt_v[sl] = typ2_v[1, sl] - typ2_v[0, sl]

    wbufs = [w0, w1, w2, w3]
    pbufs = [p0, p1]
    sem_w = [sw0, sw1, sw2, sw3]
    sem_p = [sp0, sp1]
    sem_o = [so0, so1, so2, so3]
    cp_w = [None] * NBUF
    cp_p = [None, None]
    cp_o = [None] * NBUF

    def start_pos(pc):
        cp_p[pc & 1] = pltpu.async_copy(
            pos_hbm.at[pl.ds(pbase + pc * C, C)], pbufs[pc & 1], sem_p[pc & 1])

    def start_word(s):
        pc, b = divmod(s, BATCH)
        k = s % NBUF
        cp_w[k] = pltpu.async_copy(
            word_hbm.at[widx_v.at[pl.ds(b * PPW + pc * C, C)]],
            wbufs[k], sem_w[k])

    start_pos(0)
    start_pos(1)
    for s in range(min(NBUF - 1, NSTEP)):
        start_word(s)

    for s in range(NSTEP):
        pc, b = divmod(s, BATCH)
        k = s % NBUF
        if s + NBUF - 1 < NSTEP:
            kn = (s + NBUF - 1) % NBUF
            if cp_o[kn] is not None:
                cp_o[kn].wait()
            start_word(s + NBUF - 1)
        if b == 0:
            cp_p[pc & 1].wait()
            pos_v = pbufs[pc & 1]

            @plsc.parallel_loop(0, C)
            def fold_row(i):
                @plsc.parallel_loop(0, GROUPS, unroll=8)
                def fold_grp(g):
                    sl = pl.ds(g * 16, 16)
                    pos_v[i, sl] = pos_v[i, sl] + typ2_v[0, sl]
        pos_v = pbufs[pc & 1]
        cp_w[k].wait()
        acc_v = wbufs[k]
        ttf16 = tidx_v[pl.ds(b * PPW + pc * C, C)].astype(jnp.float32)

        @plsc.parallel_loop(0, C)
        def add_row(i):
            ttf_s = lax.gather(
                ttf16, jnp.full((16, 1), i, jnp.int32), _DNUMS,
                slice_sizes=(1,),
                mode=lax.GatherScatterMode.PROMISE_IN_BOUNDS)

            @plsc.parallel_loop(0, GROUPS, unroll=16)
            def add_grp(g):
                sl = pl.ds(g * 16, 16)
                plsc.addupdate(acc_v.at[i, sl],
                               pos_v[i, sl] + ttf_s * dt_v[sl])
        if b == BATCH - 1 and pc + 2 < NPC:
            start_pos(pc + 2)
        cp_o[k] = pltpu.async_copy(
            acc_v, out_hbm.at[pl.ds(b * SEQ + pbase + pc * C, C)], sem_o[k])
    for k in range(NBUF):
        if cp_o[k] is not None:
            cp_o[k].wait()


@jax.jit
def _embed(ids, tt, word_table, type_table, pos_table):
    mesh = plsc.VectorSubcoreMesh(
        core_axis_name="c", subcore_axis_name="s", num_cores=NC, num_subcores=NS)
    k = pl.kernel(
        _embed_body,
        out_type=jax.ShapeDtypeStruct((TOK, HIDDEN), jnp.float32),
        mesh=mesh,
        scratch_types=(
            [pltpu.VMEM((BATCH * PPW,), jnp.int32)] * 2
            + [pltpu.VMEM((2, HIDDEN), jnp.float32),
               pltpu.VMEM((HIDDEN,), jnp.float32)]
            + [pltpu.VMEM((C, HIDDEN), jnp.float32)] * (NBUF + 2)
            + [pltpu.SemaphoreType.DMA] * (NBUF + 2 + NBUF)
        ),
    )
    return k(ids, tt, word_table, type_table, pos_table)


def kernel(input_ids, token_type_ids, word_table, type_table, pos_table):
    ids = input_ids.reshape(-1)
    tt = token_type_ids.reshape(-1)
    out = _embed(ids, tt, word_table, type_table, pos_table)
    return out.reshape(BATCH, SEQ, HIDDEN)


# speedup vs baseline: 1.0221x; 1.0221x over previous
import jax
import jax.numpy as jnp
from jax import lax
from jax.experimental import pallas as pl
from jax.experimental.pallas import tpu as pltpu
from jax.experimental.pallas import tpu_sc as plsc

HIDDEN = 768
BATCH = 4
SEQ = 2048
TOK = BATCH * SEQ

NC, NS = 2, 16
NW = NC * NS
PPW = SEQ // NW
C = 32
NPC = PPW // C
NSTEP = NPC * BATCH
NBUF = 3
GROUPS = HIDDEN // 16

_DNUMS = lax.GatherDimensionNumbers(
    offset_dims=(), collapsed_slice_dims=(0,), start_index_map=(0,))


def _embed_body(ids_hbm, tt_hbm, word_hbm, type_hbm, pos_hbm, out_hbm,
                widx_v, tidx_v, typ2_v, dt_v,
                w0, w1, w2, p0, p1,
                sw0, sw1, sw2, sp0, sp1, so0, so1, so2):
    wid = lax.axis_index("s") * NC + lax.axis_index("c")
    pbase = wid * PPW

    for b in range(BATCH):
        pltpu.sync_copy(ids_hbm.at[pl.ds(b * SEQ + pbase, PPW)],
                        widx_v.at[pl.ds(b * PPW, PPW)])
        pltpu.sync_copy(tt_hbm.at[pl.ds(b * SEQ + pbase, PPW)],
                        tidx_v.at[pl.ds(b * PPW, PPW)])
    pltpu.sync_copy(type_hbm, typ2_v)
    for g in range(GROUPS):
        sl = pl.ds(g * 16, 16)
        dt_v[sl] = typ2_v[1, sl] - typ2_v[0, sl]

    wbufs = [w0, w1, w2]
    pbufs = [p0, p1]
    sem_w = [sw0, sw1, sw2]
    sem_p = [sp0, sp1]
    sem_o = [so0, so1, so2]
    cp_w = [None] * NBUF
    cp_p = [None, None]
    cp_o = [None] * NBUF

    def start_pos(pc):
        cp_p[pc & 1] = pltpu.async_copy(
            pos_hbm.at[pl.ds(pbase + pc * C, C)], pbufs[pc & 1], sem_p[pc & 1])

    def start_word(s):
        pc, b = divmod(s, BATCH)
        k = s % NBUF
        cp_w[k] = pltpu.async_copy(
            word_hbm.at[widx_v.at[pl.ds(b * PPW + pc * C, C)]],
            wbufs[k], sem_w[k])

    start_pos(0)
    start_pos(1)
    for s in range(min(NBUF - 1, NSTEP)):
        start_word(s)

    for s in range(NSTEP):
        pc, b = divmod(s, BATCH)
        k = s % NBUF
        if s + NBUF - 1 < NSTEP:
            kn = (s + NBUF - 1) % NBUF
            if cp_o[kn] is not None:
                cp_o[kn].wait()
            start_word(s + NBUF - 1)
        if b == 0:
            cp_p[pc & 1].wait()
            pos_v = pbufs[pc & 1]

            @plsc.parallel_loop(0, C)
            def fold_row(i):
                @plsc.parallel_loop(0, GROUPS, unroll=8)
                def fold_grp(g):
                    sl = pl.ds(g * 16, 16)
                    pos_v[i, sl] = pos_v[i, sl] + typ2_v[0, sl]
        pos_v = pbufs[pc & 1]
        cp_w[k].wait()
        acc_v = wbufs[k]
        ttf16 = tidx_v[pl.ds(b * PPW + pc * C, C)].astype(jnp.float32)

        @plsc.parallel_loop(0, C)
        def add_row(i):
            ttf_s = lax.gather(
                ttf16, jnp.full((16, 1), i, jnp.int32), _DNUMS,
                slice_sizes=(1,),
                mode=lax.GatherScatterMode.PROMISE_IN_BOUNDS)

            @plsc.parallel_loop(0, GROUPS, unroll=8)
            def add_grp(g):
                sl = pl.ds(g * 16, 16)
                plsc.addupdate(acc_v.at[i, sl],
                               pos_v[i, sl] + ttf_s * dt_v[sl])
        if b == BATCH - 1 and pc + 2 < NPC:
            start_pos(pc + 2)
        cp_o[k] = pltpu.async_copy(
            acc_v, out_hbm.at[pl.ds(b * SEQ + pbase + pc * C, C)], sem_o[k])
    for k in range(NBUF):
        if cp_o[k] is not None:
            cp_o[k].wait()


@jax.jit
def _embed(ids, tt, word_table, type_table, pos_table):
    mesh = plsc.VectorSubcoreMesh(
        core_axis_name="c", subcore_axis_name="s", num_cores=NC, num_subcores=NS)
    k = pl.kernel(
        _embed_body,
        out_type=jax.ShapeDtypeStruct((TOK, HIDDEN), jnp.float32),
        mesh=mesh,
        scratch_types=(
            [pltpu.VMEM((BATCH * PPW,), jnp.int32)] * 2
            + [pltpu.VMEM((2, HIDDEN), jnp.float32),
               pltpu.VMEM((HIDDEN,), jnp.float32)]
            + [pltpu.VMEM((C, HIDDEN), jnp.float32)] * (NBUF + 2)
            + [pltpu.SemaphoreType.DMA] * (NBUF + 2 + NBUF)
        ),
    )
    return k(ids, tt, word_table, type_table, pos_table)


def kernel(input_ids, token_type_ids, word_table, type_table, pos_table):
    ids = input_ids.reshape(-1)
    tt = token_type_ids.reshape(-1)
    out = _embed(ids, tt, word_table, type_table, pos_table)
    return out.reshape(BATCH, SEQ, HIDDEN)


# speedup vs baseline: 1.0232x; 1.0011x over previous
import jax
import jax.numpy as jnp
from jax import lax
from jax.experimental import pallas as pl
from jax.experimental.pallas import tpu as pltpu
from jax.experimental.pallas import tpu_sc as plsc

HIDDEN = 768
BATCH = 4
SEQ = 2048
TOK = BATCH * SEQ

NC, NS = 2, 16
NW = NC * NS
PPW = SEQ // NW
C = 16
NPC = PPW // C
NSTEP = NPC * BATCH
NBUF = 6
GROUPS = HIDDEN // 16

_DNUMS = lax.GatherDimensionNumbers(
    offset_dims=(), collapsed_slice_dims=(0,), start_index_map=(0,))


def _embed_body(ids_hbm, tt_hbm, word_hbm, type_hbm, pos_hbm, out_hbm,
                widx_v, tidx_v, typ2_v, dt_v,
                w0, w1, w2, w3, w4, w5, p0, p1,
                sw0, sw1, sw2, sw3, sw4, sw5, sp0, sp1,
                so0, so1, so2, so3, so4, so5):
    wid = lax.axis_index("s") * NC + lax.axis_index("c")
    pbase = wid * PPW

    for b in range(BATCH):
        pltpu.sync_copy(ids_hbm.at[pl.ds(b * SEQ + pbase, PPW)],
                        widx_v.at[pl.ds(b * PPW, PPW)])
        pltpu.sync_copy(tt_hbm.at[pl.ds(b * SEQ + pbase, PPW)],
                        tidx_v.at[pl.ds(b * PPW, PPW)])
    pltpu.sync_copy(type_hbm, typ2_v)
    for g in range(GROUPS):
        sl = pl.ds(g * 16, 16)
        dt_v[sl] = typ2_v[1, sl] - typ2_v[0, sl]

    wbufs = [w0, w1, w2, w3, w4, w5]
    pbufs = [p0, p1]
    sem_w = [sw0, sw1, sw2, sw3, sw4, sw5]
    sem_p = [sp0, sp1]
    sem_o = [so0, so1, so2, so3, so4, so5]
    cp_w = [None] * NBUF
    cp_p = [None, None]
    cp_o = [None] * NBUF

    def start_pos(pc):
        cp_p[pc & 1] = pltpu.async_copy(
            pos_hbm.at[pl.ds(pbase + pc * C, C)], pbufs[pc & 1], sem_p[pc & 1])

    def start_word(s):
        pc, b = divmod(s, BATCH)
        k = s % NBUF
        cp_w[k] = pltpu.async_copy(
            word_hbm.at[widx_v.at[pl.ds(b * PPW + pc * C, C)]],
            wbufs[k], sem_w[k])

    start_pos(0)
    start_pos(1)
    for s in range(min(NBUF - 1, NSTEP)):
        start_word(s)

    for s in range(NSTEP):
        pc, b = divmod(s, BATCH)
        k = s % NBUF
        if s + NBUF - 1 < NSTEP:
            kn = (s + NBUF - 1) % NBUF
            if cp_o[kn] is not None:
                cp_o[kn].wait()
            start_word(s + NBUF - 1)
        if b == 0:
            cp_p[pc & 1].wait()
            pos_v = pbufs[pc & 1]

            @plsc.parallel_loop(0, C)
            def fold_row(i):
                @plsc.parallel_loop(0, GROUPS, unroll=8)
                def fold_grp(g):
                    sl = pl.ds(g * 16, 16)
                    pos_v[i, sl] = pos_v[i, sl] + typ2_v[0, sl]
        pos_v = pbufs[pc & 1]
        cp_w[k].wait()
        acc_v = wbufs[k]
        ttf16 = tidx_v[pl.ds(b * PPW + pc * C, C)].astype(jnp.float32)

        @plsc.parallel_loop(0, C)
        def add_row(i):
            ttf_s = lax.gather(
                ttf16, jnp.full((16, 1), i, jnp.int32), _DNUMS,
                slice_sizes=(1,),
                mode=lax.GatherScatterMode.PROMISE_IN_BOUNDS)

            @plsc.parallel_loop(0, GROUPS, unroll=8)
            def add_grp(g):
                sl = pl.ds(g * 16, 16)
                plsc.addupdate(acc_v.at[i, sl],
                               pos_v[i, sl] + ttf_s * dt_v[sl])
        if b == BATCH - 1 and pc + 2 < NPC:
            start_pos(pc + 2)
        cp_o[k] = pltpu.async_copy(
            acc_v, out_hbm.at[pl.ds(b * SEQ + pbase + pc * C, C)], sem_o[k])
    for k in range(NBUF):
        if cp_o[k] is not None:
            cp_o[k].wait()


@jax.jit
def _embed(ids, tt, word_table, type_table, pos_table):
    mesh = plsc.VectorSubcoreMesh(
        core_axis_name="c", subcore_axis_name="s", num_cores=NC, num_subcores=NS)
    k = pl.kernel(
        _embed_body,
        out_type=jax.ShapeDtypeStruct((TOK, HIDDEN), jnp.float32),
        mesh=mesh,
        scratch_types=(
            [pltpu.VMEM((BATCH * PPW,), jnp.int32)] * 2
            + [pltpu.VMEM((2, HIDDEN), jnp.float32),
               pltpu.VMEM((HIDDEN,), jnp.float32)]
            + [pltpu.VMEM((C, HIDDEN), jnp.float32)] * (NBUF + 2)
            + [pltpu.SemaphoreType.DMA] * (NBUF + 2 + NBUF)
        ),
    )
    return k(ids, tt, word_table, type_table, pos_table)


def kernel(input_ids, token_type_ids, word_table, type_table, pos_table):
    ids = input_ids.reshape(-1)
    tt = token_type_ids.reshape(-1)
    out = _embed(ids, tt, word_table, type_table, pos_table)
    return out.reshape(BATCH, SEQ, HIDDEN)
